# baseline (speedup 1.0000x reference)
import functools

import jax
import jax.numpy as jnp
from jax import lax
from jax.experimental import pallas as pl
from jax.experimental.pallas import tpu as pltpu
from jax.experimental.pallas import tpu_sc as plsc

P = 0.5

_info = plsc.get_sparse_core_info()
_NC, _NS, _L = _info.num_cores, _info.num_subcores, _info.num_lanes
_NW = _NC * _NS


def _make_kernel(bn):
    ppw = bn // _NW
    cpts = 8192
    nchunk = ppw // cpts
    spans = cpts // _L
    cov_rows = cpts * 9 // 128
    flip_rows = cpts // 128
    mesh = plsc.VectorSubcoreMesh(core_axis_name="c", subcore_axis_name="s")

    @functools.partial(
        pl.kernel,
        mesh=mesh,
        out_type=jax.ShapeDtypeStruct((bn * 9 // 128, 128), jnp.float32),
        scratch_types=[
            pltpu.VMEM((cov_rows, 128), jnp.float32),
            pltpu.VMEM((flip_rows, 128), jnp.float32),
            pltpu.VMEM((144,), jnp.float32),
            pltpu.VMEM((144,), jnp.int32),
        ],
    )
    def k(cov_hbm, flip_hbm, droppat_hbm, idxpat_hbm, out_hbm,
          cov_v, flip_v, droppat_v, idxpat_v):
        wid = lax.axis_index("s") * _NC + lax.axis_index("c")

        pltpu.sync_copy(droppat_hbm, droppat_v)
        pltpu.sync_copy(idxpat_hbm, idxpat_v)

        dropv = [droppat_v[pl.ds(16 * ph, 16)] for ph in range(9)]
        idxv = [idxpat_v[pl.ds(16 * ph, 16)] for ph in range(9)]

        def span_body(s, _):
            f = flip_v[s >> 3, pl.ds((s & 7) * 16, 16)]
            t0 = s * 9
            for ph in range(9):
                t = t0 + ph
                fv = lax.gather(
                    f, idxv[ph][:, None],
                    lax.GatherDimensionNumbers(
                        offset_dims=(), collapsed_slice_dims=(0,),
                        start_index_map=(0,)),
                    slice_sizes=(1,),
                    mode=lax.GatherScatterMode.PROMISE_IN_BOUNDS)
                cv = cov_v[t >> 3, pl.ds((t & 7) * 16, 16)]
                cov_v[t >> 3, pl.ds((t & 7) * 16, 16)] = jnp.where(
                    fv >= P, cv, dropv[ph])
            return 0

        def chunk_body(c, _):
            crow = pl.multiple_of((wid * ppw + c * cpts) * 9 // 128, 8)
            frow = pl.multiple_of((wid * ppw + c * cpts) // 128, 8)
            pltpu.sync_copy(flip_hbm.at[pl.ds(frow, flip_rows)], flip_v)
            pltpu.sync_copy(cov_hbm.at[pl.ds(crow, cov_rows)], cov_v)
            lax.fori_loop(0, spans, span_body, 0)
            pltpu.sync_copy(cov_v, out_hbm.at[pl.ds(crow, cov_rows)])
            return 0

        lax.fori_loop(0, nchunk, chunk_body, 0)

    return k


def kernel(cov, drop_cov, flip):
    b, n, d, _ = cov.shape
    bn = b * n
    cov_flat = cov.reshape(bn * d * d // 128, 128)
    flip2 = flip.reshape(bn // 128, 128)
    j = jnp.arange(144, dtype=jnp.int32)
    idx_pat = j // 9
    drop_pat = drop_cov.reshape(9)[j % 9]
    out = _make_kernel(bn)(cov_flat, flip2, drop_pat, idx_pat)
    return out.reshape(b, n, d, d)


# speedup vs baseline: 201.3086x; 201.3086x over previous
import functools

import jax
import jax.numpy as jnp
from jax import lax
from jax.experimental import pallas as pl
from jax.experimental.pallas import tpu as pltpu
from jax.experimental.pallas import tpu_sc as plsc

P = 0.5

_info = plsc.get_sparse_core_info()
_NC, _NS, _L = _info.num_cores, _info.num_subcores, _info.num_lanes
_NW = _NC * _NS


def _make_kernel(b, n):
    ntr = b // 8
    ntc = n // 128
    tc_per_w = ntc // _NW
    tcg = 8
    ngrp = tc_per_w // tcg
    mesh = plsc.VectorSubcoreMesh(core_axis_name="c", subcore_axis_name="s")

    @functools.partial(
        pl.kernel,
        mesh=mesh,
        out_type=jax.ShapeDtypeStruct((9, ntr, ntc, 8, 128), jnp.float32),
        scratch_types=[
            pltpu.VMEM((9, tcg, 8, 128), jnp.float32),
            pltpu.VMEM((8, tcg, 128), jnp.float32),
            pltpu.VMEM((144,), jnp.float32),
        ],
    )
    def k(cov_hbm, flip_hbm, droppat_hbm, out_hbm, cov_v, flip_v, droppat_v):
        wid = lax.axis_index("s") * _NC + lax.axis_index("c")

        pltpu.sync_copy(droppat_hbm, droppat_v)
        dropv = [droppat_v[pl.ds(16 * p, 16)] for p in range(9)]

        def sel_body(it, _):
            tcl = it >> 6
            r = (it >> 3) & 7
            j = it & 7
            f = flip_v[r, tcl, pl.ds(j * 16, 16)]
            keep = f >= P
            for p in range(9):
                cv = cov_v[p, tcl, r, pl.ds(j * 16, 16)]
                cov_v[p, tcl, r, pl.ds(j * 16, 16)] = jnp.where(
                    keep, cv, dropv[p])
            return 0

        def chunk_body(g, _):
            tr = g // ngrp
            tc0 = wid * tc_per_w + (g % ngrp) * tcg
            pltpu.sync_copy(flip_hbm.at[tr, :, pl.ds(tc0, tcg), :], flip_v)
            pltpu.sync_copy(cov_hbm.at[:, tr, pl.ds(tc0, tcg), :, :], cov_v)
            lax.fori_loop(0, tcg * 64, sel_body, 0)
            pltpu.sync_copy(cov_v, out_hbm.at[:, tr, pl.ds(tc0, tcg), :, :])
            return 0

        lax.fori_loop(0, ntr * ngrp, chunk_body, 0)

    return k


def kernel(cov, drop_cov, flip):
    b, n, d, _ = cov.shape
    cov5 = (cov.transpose(2, 3, 0, 1)
               .reshape(d * d, b // 8, 8, n // 128, 128)
               .transpose(0, 1, 3, 2, 4))
    flip4 = flip.reshape(b // 8, 8, n // 128, 128)
    drop_pat = jnp.repeat(drop_cov.reshape(d * d), 16)
    out = _make_kernel(b, n)(cov5, flip4, drop_pat)
    out = (out.transpose(0, 1, 3, 2, 4)
              .reshape(d, d, b, n)
              .transpose(2, 3, 0, 1))
    return out


# speedup vs baseline: 315.1424x; 1.5655x over previous
import functools

import jax
import jax.numpy as jnp
from jax import lax
from jax.experimental import pallas as pl
from jax.experimental.pallas import tpu as pltpu
from jax.experimental.pallas import tpu_sc as plsc

P = 0.5

_info = plsc.get_sparse_core_info()
_NC, _NS, _L = _info.num_cores, _info.num_subcores, _info.num_lanes
_NW = _NC * _NS


def _make_kernel(b, n):
    ntr = b // 8
    ntc = n // 128
    tc_per_w = ntc // _NW
    tcg = 4
    ngrp = tc_per_w // tcg
    nchunk = ntr * ngrp
    mesh = plsc.VectorSubcoreMesh(core_axis_name="c", subcore_axis_name="s")

    @functools.partial(
        pl.kernel,
        mesh=mesh,
        out_type=jax.ShapeDtypeStruct((9, ntr, ntc, 8, 128), jnp.float32),
        scratch_types=[
            pltpu.VMEM((9, tcg, 8, 128), jnp.float32),
            pltpu.VMEM((9, tcg, 8, 128), jnp.float32),
            pltpu.VMEM((8, tcg, 128), jnp.float32),
            pltpu.VMEM((8, tcg, 128), jnp.float32),
            pltpu.VMEM((144,), jnp.float32),
            pltpu.SemaphoreType.DMA,
            pltpu.SemaphoreType.DMA,
            pltpu.SemaphoreType.DMA,
            pltpu.SemaphoreType.DMA,
        ],
    )
    def k(cov_hbm, flip_hbm, droppat_hbm, out_hbm,
          cov_v0, cov_v1, flip_v0, flip_v1, droppat_v,
          sin0, sin1, sout0, sout1):
        wid = lax.axis_index("s") * _NC + lax.axis_index("c")
        covb = (cov_v0, cov_v1)
        flipb = (flip_v0, flip_v1)
        sinb = (sin0, sin1)
        soutb = (sout0, sout1)

        pltpu.sync_copy(droppat_hbm, droppat_v)
        dropv = [droppat_v[pl.ds(16 * p, 16)] for p in range(9)]

        def loc(q):
            tr = q // ngrp
            tc0 = wid * tc_per_w + (q % ngrp) * tcg
            return tr, tc0

        def in_copies(q, h):
            tr, tc0 = loc(q)
            fd = pltpu.make_async_copy(
                flip_hbm.at[tr, :, pl.ds(tc0, tcg), :], flipb[h], sinb[h])
            cd = pltpu.make_async_copy(
                cov_hbm.at[:, tr, pl.ds(tc0, tcg), :, :], covb[h], sinb[h])
            return fd, cd

        def out_copy(q, h):
            tr, tc0 = loc(q)
            return pltpu.make_async_copy(
                covb[h], out_hbm.at[:, tr, pl.ds(tc0, tcg), :, :], soutb[h])

        def start_in(q, h):
            fd, cd = in_copies(q, h)
            fd.start()
            cd.start()

        def wait_in(q, h):
            fd, cd = in_copies(q, h)
            fd.wait()
            cd.wait()

        def compute(h):
            cov_v = covb[h]
            flip_v = flipb[h]

            def sel_body(it, _):
                tcl = it >> 6
                r = (it >> 3) & 7
                j = it & 7
                f = flip_v[r, tcl, pl.ds(j * 16, 16)]
                keep = f >= P
                for p in range(9):
                    cv = cov_v[p, tcl, r, pl.ds(j * 16, 16)]
                    cov_v[p, tcl, r, pl.ds(j * 16, 16)] = jnp.where(
                        keep, cv, dropv[p])
                return 0

            lax.fori_loop(0, tcg * 64, sel_body, 0)

        start_in(0, 0)

        def body(g2, _):
            g = g2 * 2

            @pl.when(g2 >= 1)
            def _():
                out_copy(g - 2, 0).wait()
            wait_in(g, 0)
            start_in(g + 1, 1)
            compute(0)
            out_copy(g, 0).start()

            @pl.when(g2 >= 1)
            def _():
                out_copy(g - 1, 1).wait()
            wait_in(g + 1, 1)

            @pl.when(g2 < (nchunk // 2) - 1)
            def _():
                start_in(g + 2, 0)
            compute(1)
            out_copy(g + 1, 1).start()
            return 0

        lax.fori_loop(0, nchunk // 2, body, 0)
        out_copy(nchunk - 2, 0).wait()
        out_copy(nchunk - 1, 1).wait()

    return k


def kernel(cov, drop_cov, flip):
    b, n, d, _ = cov.shape
    cov5 = (cov.transpose(2, 3, 0, 1)
               .reshape(d * d, b // 8, 8, n // 128, 128)
               .transpose(0, 1, 3, 2, 4))
    flip4 = flip.reshape(b // 8, 8, n // 128, 128)
    drop_pat = jnp.repeat(drop_cov.reshape(d * d), 16)
    out = _make_kernel(b, n)(cov5, flip4, drop_pat)
    out = (out.transpose(0, 1, 3, 2, 4)
              .reshape(d, d, b, n)
              .transpose(2, 3, 0, 1))
    return out
